# baseline (speedup 1.0000x reference)
import functools

import jax
import jax.numpy as jnp
from jax import lax
from jax.experimental import pallas as pl
from jax.experimental.pallas import tpu as pltpu

N_PTS = 8192
KNN = 20
ROWS = 1024
CHUNK = 1024
NCH_MAX = N_PTS // CHUNK
PTILE = 128
N_EDGE = N_PTS * KNN
EPS = 1e-5
BIG_I = 2**30
INF_F = float('inf')


def _bf(t):
    return t.astype(jnp.bfloat16)


def _mm(a, b):
    return lax.dot_general(_bf(a), _bf(b), (((1,), (0,)), ((), ())),
                           preferred_element_type=jnp.float32)


def _dist_topk_body(c0_ref, nch_ref, xr_ref, xc_ref, sqr_ref, sqc_ref,
                    bcol_ref, brow_ref, idx_ref, dist_scr):
    i = pl.program_id(0)
    c0 = c0_ref[i]
    nch = nch_ref[i]
    cj0 = c0 // CHUNK
    xt = xr_ref[i]
    sqt = sqr_ref[i]
    bt = bcol_ref[i]

    def fill(j, carry):
        xc = xc_ref[cj0 + j]
        g = lax.dot_general(xt, xc, (((1,), (1,)), ((), ())),
                            preferred_element_type=jnp.float32)
        dt = (sqt + sqc_ref[cj0 + j]) - 2.0 * g
        dt = jnp.where(bt == brow_ref[cj0 + j], dt, INF_F)
        dist_scr[j] = dt
        return carry

    lax.fori_loop(0, nch, fill, 0)

    def minpass(j, m):
        return jnp.minimum(m, jnp.min(dist_scr[j], axis=1, keepdims=True))

    m = lax.fori_loop(0, nch, minpass, jnp.full((ROWS, 1), INF_F))

    lane = lax.broadcasted_iota(jnp.int32, (ROWS, CHUNK), 1)
    for r in range(KNN):
        mr = m

        def argpass(j, best):
            ch = dist_scr[j]
            cand = jnp.where(ch == mr, lane + j * CHUNK, BIG_I)
            return jnp.minimum(best, jnp.min(cand, axis=1, keepdims=True))

        idxc = lax.fori_loop(0, nch, argpass,
                             jnp.full((ROWS, 1), BIG_I, jnp.int32))
        idx_ref[:, r:r + 1] = idxc + c0
        if r < KNN - 1:
            def maskpass(j, m2):
                ch = dist_scr[j]
                ch = jnp.where(lane + j * CHUNK == idxc, INF_F, ch)
                dist_scr[j] = ch
                return jnp.minimum(m2, jnp.min(ch, axis=1, keepdims=True))

            m = lax.fori_loop(0, nch, maskpass, jnp.full((ROWS, 1), INF_F))


def _dist_topk(x, bcol, brow, c0s, nchs):
    n, d = x.shape
    nrt = n // ROWS
    xb = _bf(x)
    sq = jnp.sum(x * x, axis=1)
    grid_spec = pltpu.PrefetchScalarGridSpec(
        num_scalar_prefetch=2,
        grid=(nrt,),
        in_specs=[
            pl.BlockSpec((nrt, ROWS, d), lambda i, *_: (0, 0, 0)),
            pl.BlockSpec((NCH_MAX, CHUNK, d), lambda i, *_: (0, 0, 0)),
            pl.BlockSpec((nrt, ROWS, 1), lambda i, *_: (0, 0, 0)),
            pl.BlockSpec((NCH_MAX, 1, CHUNK), lambda i, *_: (0, 0, 0)),
            pl.BlockSpec((nrt, ROWS, 1), lambda i, *_: (0, 0, 0)),
            pl.BlockSpec((NCH_MAX, 1, CHUNK), lambda i, *_: (0, 0, 0)),
        ],
        out_specs=pl.BlockSpec((ROWS, 32), lambda i, *_: (i, 0)),
        scratch_shapes=[pltpu.VMEM((NCH_MAX, ROWS, CHUNK), jnp.float32)],
    )
    idx = pl.pallas_call(
        _dist_topk_body,
        grid_spec=grid_spec,
        out_shape=jax.ShapeDtypeStruct((n, 32), jnp.int32),
    )(c0s, nchs,
      xb.reshape(nrt, ROWS, d),
      xb.reshape(NCH_MAX, CHUNK, d),
      sq.reshape(nrt, ROWS, 1),
      sq.reshape(NCH_MAX, 1, CHUNK),
      bcol.reshape(nrt, ROWS, 1),
      brow.reshape(NCH_MAX, 1, CHUNK))
    return idx[:, :KNN]


SC_NW = 32
SC_CH = 128


def _gather_rows(table, idx_flat):
    from jax.experimental.pallas import tpu_sc as plsc
    v, d = table.shape
    b = idx_flat.shape[0]
    bw = b // SC_NW
    nch = bw // SC_CH
    mesh = plsc.VectorSubcoreMesh(core_axis_name="c", subcore_axis_name="s")

    @functools.partial(
        pl.kernel, mesh=mesh,
        out_type=jax.ShapeDtypeStruct((b, d), jnp.float32),
        scratch_types=[
            pltpu.VMEM((SC_CH,), jnp.int32),
            pltpu.VMEM((SC_CH, d), jnp.float32),
            pltpu.SemaphoreType.DMA,
        ],
    )
    def k(table_hbm, idx_hbm, out_hbm, idx_v, rows_v, sem):
        wid = lax.axis_index("s") * 2 + lax.axis_index("c")
        base = wid * bw

        def body(j, carry):
            off = base + j * SC_CH
            pltpu.sync_copy(idx_hbm.at[pl.ds(off, SC_CH)], idx_v)
            pltpu.async_copy(table_hbm.at[idx_v], rows_v, sem).wait()
            pltpu.sync_copy(rows_v, out_hbm.at[pl.ds(off, SC_CH)])
            return carry

        lax.fori_loop(0, nch, body, 0)

    return k(table, idx_flat)


def _edge_h1(d, dg, xg_ref, x_ref, w1a_ref, b1_ref, w1b_ref):
    xi = x_ref[...][:, :dg]
    ua = _mm(xi[:, :d], w1a_ref[...]) + b1_ref[...]
    e2 = xg_ref[...][:, :, :dg] - xi[None, :, :]
    p = _mm(e2.reshape(KNN * PTILE, dg), w1b_ref[...])
    return ua[None, :, :] + p.reshape(KNN, PTILE, ua.shape[-1])


def _bn_fin(ne, sums, g, be):
    mean = sums[0:1, :] / ne
    var = sums[1:2, :] / ne - mean * mean
    a = g * lax.rsqrt(var + EPS)
    return a, be - mean * a


def _acc_stats(i, nt, ne, z2, g_ref, be_ref, a_ref, c_ref, sums_scr):
    upd = jnp.stack([jnp.sum(z2, axis=0), jnp.sum(z2 * z2, axis=0)], axis=0)

    @pl.when(i == 0)
    def _():
        sums_scr[...] = jnp.zeros_like(sums_scr)

    sums_scr[...] += upd

    @pl.when(i == nt - 1)
    def _():
        a, c = _bn_fin(ne, sums_scr[...], g_ref[...], be_ref[...])
        a_ref[...] = a
        c_ref[...] = c


def _s1_body(nt, d, dg, xg_ref, x_ref, w1a_ref, b1_ref, w1b_ref,
             g_ref, be_ref, a_ref, c_ref, sums_scr):
    i = pl.program_id(0)
    h1 = _edge_h1(d, dg, xg_ref, x_ref, w1a_ref, b1_ref, w1b_ref)
    h1f = h1.reshape(KNN * PTILE, h1.shape[-1])
    _acc_stats(i, nt, float(N_EDGE), h1f, g_ref, be_ref, a_ref, c_ref,
               sums_scr)


def _s1(xg, x, d, dg, w1a, b1, w1b, g, be):
    n = x.shape[0]
    h = w1a.shape[1]
    nt = n // PTILE
    return pl.pallas_call(
        functools.partial(_s1_body, nt, d, dg),
        grid=(nt,),
        in_specs=[
            pl.BlockSpec((KNN, PTILE, 128), lambda i: (0, i, 0)),
            pl.BlockSpec((PTILE, 128), lambda i: (i, 0)),
            pl.BlockSpec((d, h), lambda i: (0, 0)),
            pl.BlockSpec((1, h), lambda i: (0, 0)),
            pl.BlockSpec((dg, h), lambda i: (0, 0)),
            pl.BlockSpec((1, h), lambda i: (0, 0)),
            pl.BlockSpec((1, h), lambda i: (0, 0)),
        ],
        out_specs=(pl.BlockSpec((1, h), lambda i: (0, 0)),
                   pl.BlockSpec((1, h), lambda i: (0, 0))),
        out_shape=(jax.ShapeDtypeStruct((1, h), jnp.float32),
                   jax.ShapeDtypeStruct((1, h), jnp.float32)),
        scratch_shapes=[pltpu.VMEM((2, h), jnp.float32)],
    )(xg, x, w1a, b1.reshape(1, h), w1b, g.reshape(1, h), be.reshape(1, h))


def _s2_body(nt, d, dg, xg_ref, x_ref, w1a_ref, b1_ref, w1b_ref,
             a1_ref, c1_ref, w2_ref, b2_ref, g_ref, be_ref,
             a_ref, c_ref, sums_scr):
    i = pl.program_id(0)
    h1 = _edge_h1(d, dg, xg_ref, x_ref, w1a_ref, b1_ref, w1b_ref)
    t = jnp.maximum(h1 * a1_ref[...] + c1_ref[...], 0.0)
    t2 = t.reshape(KNN * PTILE, t.shape[-1])
    z = _mm(t2, w2_ref[...]) + b2_ref[...]
    _acc_stats(i, nt, float(N_EDGE), z, g_ref, be_ref, a_ref, c_ref,
               sums_scr)


def _s2(xg, x, d, dg, w1a, b1, w1b, a1, c1, w2, b2, g2, be2):
    n = x.shape[0]
    h = w1a.shape[1]
    h2 = w2.shape[1]
    nt = n // PTILE
    return pl.pallas_call(
        functools.partial(_s2_body, nt, d, dg),
        grid=(nt,),
        in_specs=[
            pl.BlockSpec((KNN, PTILE, 128), lambda i: (0, i, 0)),
            pl.BlockSpec((PTILE, 128), lambda i: (i, 0)),
            pl.BlockSpec((d, h), lambda i: (0, 0)),
            pl.BlockSpec((1, h), lambda i: (0, 0)),
            pl.BlockSpec((dg, h), lambda i: (0, 0)),
            pl.BlockSpec((1, h), lambda i: (0, 0)),
            pl.BlockSpec((1, h), lambda i: (0, 0)),
            pl.BlockSpec((h, h2), lambda i: (0, 0)),
            pl.BlockSpec((1, h2), lambda i: (0, 0)),
            pl.BlockSpec((1, h2), lambda i: (0, 0)),
            pl.BlockSpec((1, h2), lambda i: (0, 0)),
        ],
        out_specs=(pl.BlockSpec((1, h2), lambda i: (0, 0)),
                   pl.BlockSpec((1, h2), lambda i: (0, 0))),
        out_shape=(jax.ShapeDtypeStruct((1, h2), jnp.float32),
                   jax.ShapeDtypeStruct((1, h2), jnp.float32)),
        scratch_shapes=[pltpu.VMEM((2, h2), jnp.float32)],
    )(xg, x, w1a, b1.reshape(1, h), w1b, a1, c1, w2, b2.reshape(1, h2),
      g2.reshape(1, h2), be2.reshape(1, h2))


def _emax_body(d, dg, xg_ref, x_ref, w1a_ref, b1_ref, w1b_ref,
               a1_ref, c1_ref, w2_ref, b2_ref, a2_ref, c2_ref, out_ref):
    h1 = _edge_h1(d, dg, xg_ref, x_ref, w1a_ref, b1_ref, w1b_ref)
    t = jnp.maximum(h1 * a1_ref[...] + c1_ref[...], 0.0)
    t2 = t.reshape(KNN * PTILE, t.shape[-1])
    z = _mm(t2, w2_ref[...]) + b2_ref[...]
    z = jnp.maximum(z * a2_ref[...] + c2_ref[...], 0.0)
    z3 = z.reshape(KNN, PTILE, z.shape[-1])
    out_ref[...] = jnp.max(z3, axis=0)


def _emax(xg, x, d, dg, w1a, b1, w1b, a1, c1, w2, b2, a2, c2):
    n = x.shape[0]
    h = w1a.shape[1]
    h2 = w2.shape[1]
    nt = n // PTILE
    return pl.pallas_call(
        functools.partial(_emax_body, d, dg),
        grid=(nt,),
        in_specs=[
            pl.BlockSpec((KNN, PTILE, 128), lambda i: (0, i, 0)),
            pl.BlockSpec((PTILE, 128), lambda i: (i, 0)),
            pl.BlockSpec((d, h), lambda i: (0, 0)),
            pl.BlockSpec((1, h), lambda i: (0, 0)),
            pl.BlockSpec((dg, h), lambda i: (0, 0)),
            pl.BlockSpec((1, h), lambda i: (0, 0)),
            pl.BlockSpec((1, h), lambda i: (0, 0)),
            pl.BlockSpec((h, h2), lambda i: (0, 0)),
            pl.BlockSpec((1, h2), lambda i: (0, 0)),
            pl.BlockSpec((1, h2), lambda i: (0, 0)),
            pl.BlockSpec((1, h2), lambda i: (0, 0)),
        ],
        out_specs=pl.BlockSpec((PTILE, h2), lambda i: (i, 0)),
        out_shape=jax.ShapeDtypeStruct((n, h2), jnp.float32),
    )(xg, x, w1a, b1.reshape(1, h), w1b, a1, c1, w2, b2.reshape(1, h2),
      a2, c2)


def _c1_body(nt, x1_ref, x2_ref, w_ref, b_ref, g_ref, be_ref,
             hc_ref, a_ref, c_ref, sums_scr):
    i = pl.program_id(0)
    xc = jnp.concatenate([x1_ref[...], x2_ref[...]], axis=1)
    hc = _mm(xc, w_ref[...]) + b_ref[...]
    hc_ref[...] = hc
    _acc_stats(i, nt, float(N_PTS), hc, g_ref, be_ref, a_ref, c_ref,
               sums_scr)


def _c1(x1, x2, w, b, g, be):
    n = x1.shape[0]
    h1 = x1.shape[1]
    h2 = x2.shape[1]
    ho = w.shape[1]
    nt = n // PTILE
    return pl.pallas_call(
        functools.partial(_c1_body, nt),
        grid=(nt,),
        in_specs=[
            pl.BlockSpec((PTILE, h1), lambda i: (i, 0)),
            pl.BlockSpec((PTILE, h2), lambda i: (i, 0)),
            pl.BlockSpec((h1 + h2, ho), lambda i: (0, 0)),
            pl.BlockSpec((1, ho), lambda i: (0, 0)),
            pl.BlockSpec((1, ho), lambda i: (0, 0)),
            pl.BlockSpec((1, ho), lambda i: (0, 0)),
        ],
        out_specs=(pl.BlockSpec((PTILE, ho), lambda i: (i, 0)),
                   pl.BlockSpec((1, ho), lambda i: (0, 0)),
                   pl.BlockSpec((1, ho), lambda i: (0, 0))),
        out_shape=(jax.ShapeDtypeStruct((n, ho), jnp.float32),
                   jax.ShapeDtypeStruct((1, ho), jnp.float32),
                   jax.ShapeDtypeStruct((1, ho), jnp.float32)),
        scratch_shapes=[pltpu.VMEM((2, ho), jnp.float32)],
    )(x1, x2, w, b.reshape(1, ho), g.reshape(1, ho), be.reshape(1, ho))


def _c2_body(hc_ref, a_ref, c_ref, w2_ref, b2_ref, out_ref):
    t = jnp.maximum(hc_ref[...] * a_ref[...] + c_ref[...], 0.0)
    out_ref[...] = _mm(t, w2_ref[...]) + b2_ref[...]


def _c2(hc, a, c, w2, b2):
    n, h = hc.shape
    nt = n // PTILE
    return pl.pallas_call(
        _c2_body,
        grid=(nt,),
        in_specs=[
            pl.BlockSpec((PTILE, h), lambda i: (i, 0)),
            pl.BlockSpec((1, h), lambda i: (0, 0)),
            pl.BlockSpec((1, h), lambda i: (0, 0)),
            pl.BlockSpec((h, 1), lambda i: (0, 0)),
            pl.BlockSpec((1, 1), lambda i: (0, 0)),
        ],
        out_specs=pl.BlockSpec((PTILE, 1), lambda i: (i, 0)),
        out_shape=jax.ShapeDtypeStruct((n, 1), jnp.float32),
    )(hc, a, c, w2, b2.reshape(1, 1))


def _edgeconv(x, xg_src, dg, bcol, brow, c0s, nchs,
              w1, b1, g1, be1, w2, b2, g2, be2):
    n, d = x.shape
    h = w1.shape[1]
    w1a = w1[:d, :]
    w1b_p = jnp.zeros((dg, h), jnp.float32).at[:d, :].set(w1[d:, :])
    idx = _dist_topk(x, bcol, brow, c0s, nchs)
    idxf = jnp.transpose(idx).reshape(-1)
    xg = _gather_rows(xg_src, idxf).reshape(KNN, n, 128)
    a1, c1 = _s1(xg, xg_src, d, dg, w1a, b1, w1b_p, g1, be1)
    a2, c2 = _s2(xg, xg_src, d, dg, w1a, b1, w1b_p, a1, c1, w2, b2, g2, be2)
    return _emax(xg, xg_src, d, dg, w1a, b1, w1b_p, a1, c1, w2, b2, a2, c2)


def kernel(x, batch,
           c1_w1, c1_b1, c1_g1, c1_be1, c1_w2, c1_b2, c1_g2, c1_be2,
           c2_w1, c2_b1, c2_g1, c2_be1, c2_w2, c2_b2, c2_g2, c2_be2,
           cl_w1, cl_b1, cl_g1, cl_be1, cl_w2, cl_b2):
    n = x.shape[0]
    bi = batch.astype(jnp.int32)
    bcol = bi.reshape(n, 1)
    brow = bi.reshape(1, n)
    seg = jnp.searchsorted(bi, jnp.arange(9, dtype=jnp.int32), side='left')
    seg = seg.astype(jnp.int32)
    rt = jnp.arange(n // ROWS, dtype=jnp.int32)
    blo = bi[rt * ROWS]
    bhi = bi[rt * ROWS + (ROWS - 1)]
    wlo = seg[blo]
    whi = seg[bhi + 1]
    c0s = (wlo // CHUNK) * CHUNK
    nchs = (whi + CHUNK - 1 - c0s) // CHUNK

    x_pad = jnp.zeros((n, 128), jnp.float32).at[:, :x.shape[1]].set(x)
    x1 = _edgeconv(x, x_pad, 16, bcol, brow, c0s, nchs,
                   c1_w1, c1_b1, c1_g1, c1_be1, c1_w2, c1_b2, c1_g2, c1_be2)
    x1_pad = jnp.zeros((n, 128), jnp.float32).at[:, :x1.shape[1]].set(x1)
    x2 = _edgeconv(x1, x1_pad, 64, bcol, brow, c0s, nchs,
                   c2_w1, c2_b1, c2_g1, c2_be1, c2_w2, c2_b2, c2_g2, c2_be2)
    hc, ac, cc = _c1(x1, x2, cl_w1, cl_b1, cl_g1, cl_be1)
    return _c2(hc, ac, cc, cl_w2, cl_b2)


# speedup vs baseline: 1.0890x; 1.0890x over previous
import functools

import jax
import jax.numpy as jnp
from jax import lax
from jax.experimental import pallas as pl
from jax.experimental.pallas import tpu as pltpu

N_PTS = 8192
KNN = 20
ROWS = 512
CHUNK = 1024
NCH_MAX = N_PTS // CHUNK
PTILE = 128
N_EDGE = N_PTS * KNN
EPS = 1e-5
BIG_I = 2**30
INF_F = float('inf')


def _bf(t):
    return t.astype(jnp.bfloat16)


def _mm(a, b):
    return lax.dot_general(_bf(a), _bf(b), (((1,), (0,)), ((), ())),
                           preferred_element_type=jnp.float32)


def _dist_topk_body(c0_ref, nch_ref, xr_ref, xc_ref, sqr_ref, sqc_ref,
                    bcol_ref, brow_ref, idx_ref, dist_scr):
    i = pl.program_id(0)
    c0 = c0_ref[i]
    nch = nch_ref[i]
    cj0 = c0 // CHUNK
    xt = xr_ref[i]
    sqt = sqr_ref[i]
    bt = bcol_ref[i]

    def fill(j, carry):
        xc = xc_ref[cj0 + j]
        g = lax.dot_general(xt, xc, (((1,), (1,)), ((), ())),
                            preferred_element_type=jnp.float32)
        dt = (sqt + sqc_ref[cj0 + j]) - 2.0 * g
        dt = jnp.where(bt == brow_ref[cj0 + j], dt, INF_F)
        dist_scr[j] = dt
        return carry

    lax.fori_loop(0, nch, fill, 0)

    def minpass(j, m):
        return jnp.minimum(m, jnp.min(dist_scr[j], axis=1, keepdims=True))

    m = lax.fori_loop(0, nch, minpass, jnp.full((ROWS, 1), INF_F))

    lane = lax.broadcasted_iota(jnp.int32, (ROWS, CHUNK), 1)
    for r in range(KNN):
        mr = m

        def argpass(j, best):
            ch = dist_scr[j]
            cand = jnp.where(ch == mr, lane + j * CHUNK, BIG_I)
            return jnp.minimum(best, jnp.min(cand, axis=1, keepdims=True))

        idxc = lax.fori_loop(0, nch, argpass,
                             jnp.full((ROWS, 1), BIG_I, jnp.int32))
        idx_ref[:, r:r + 1] = idxc + c0
        if r < KNN - 1:
            def maskpass(j, m2):
                ch = dist_scr[j]
                ch = jnp.where(lane + j * CHUNK == idxc, INF_F, ch)
                dist_scr[j] = ch
                return jnp.minimum(m2, jnp.min(ch, axis=1, keepdims=True))

            m = lax.fori_loop(0, nch, maskpass, jnp.full((ROWS, 1), INF_F))


def _dist_topk(x, bcol, brow, c0s, nchs):
    n, d = x.shape
    nrt = n // ROWS
    xb = _bf(x)
    sq = jnp.sum(x * x, axis=1)
    grid_spec = pltpu.PrefetchScalarGridSpec(
        num_scalar_prefetch=2,
        grid=(nrt,),
        in_specs=[
            pl.BlockSpec((nrt, ROWS, d), lambda i, *_: (0, 0, 0)),
            pl.BlockSpec((NCH_MAX, CHUNK, d), lambda i, *_: (0, 0, 0)),
            pl.BlockSpec((nrt, ROWS, 1), lambda i, *_: (0, 0, 0)),
            pl.BlockSpec((NCH_MAX, 1, CHUNK), lambda i, *_: (0, 0, 0)),
            pl.BlockSpec((nrt, ROWS, 1), lambda i, *_: (0, 0, 0)),
            pl.BlockSpec((NCH_MAX, 1, CHUNK), lambda i, *_: (0, 0, 0)),
        ],
        out_specs=pl.BlockSpec((ROWS, 32), lambda i, *_: (i, 0)),
        scratch_shapes=[pltpu.VMEM((NCH_MAX, ROWS, CHUNK), jnp.float32)],
    )
    idx = pl.pallas_call(
        _dist_topk_body,
        grid_spec=grid_spec,
        out_shape=jax.ShapeDtypeStruct((n, 32), jnp.int32),
    )(c0s, nchs,
      xb.reshape(nrt, ROWS, d),
      xb.reshape(NCH_MAX, CHUNK, d),
      sq.reshape(nrt, ROWS, 1),
      sq.reshape(NCH_MAX, 1, CHUNK),
      bcol.reshape(nrt, ROWS, 1),
      brow.reshape(NCH_MAX, 1, CHUNK))
    return idx[:, :KNN]


SC_NW = 32
SC_CH = 128


def _gather_rows(table, idx_flat):
    from jax.experimental.pallas import tpu_sc as plsc
    v, d = table.shape
    b = idx_flat.shape[0]
    bw = b // SC_NW
    nch = bw // SC_CH
    mesh = plsc.VectorSubcoreMesh(core_axis_name="c", subcore_axis_name="s")

    @functools.partial(
        pl.kernel, mesh=mesh,
        out_type=jax.ShapeDtypeStruct((b, d), jnp.float32),
        scratch_types=[
            pltpu.VMEM((SC_CH,), jnp.int32),
            pltpu.VMEM((SC_CH, d), jnp.float32),
            pltpu.SemaphoreType.DMA,
        ],
    )
    def k(table_hbm, idx_hbm, out_hbm, idx_v, rows_v, sem):
        wid = lax.axis_index("s") * 2 + lax.axis_index("c")
        base = wid * bw

        def body(j, carry):
            off = base + j * SC_CH
            pltpu.sync_copy(idx_hbm.at[pl.ds(off, SC_CH)], idx_v)
            pltpu.async_copy(table_hbm.at[idx_v], rows_v, sem).wait()
            pltpu.sync_copy(rows_v, out_hbm.at[pl.ds(off, SC_CH)])
            return carry

        lax.fori_loop(0, nch, body, 0)

    return k(table, idx_flat)


def _edge_h1(d, dg, xg_ref, x_ref, w1a_ref, b1_ref, w1b_ref):
    xi = x_ref[...][:, :dg]
    ua = _mm(xi[:, :d], w1a_ref[...]) + b1_ref[...]
    e2 = xg_ref[...][:, :, :dg] - xi[None, :, :]
    p = _mm(e2.reshape(KNN * PTILE, dg), w1b_ref[...])
    return ua[None, :, :] + p.reshape(KNN, PTILE, ua.shape[-1])


def _bn_fin(ne, sums, g, be):
    mean = sums[0:1, :] / ne
    var = sums[1:2, :] / ne - mean * mean
    a = g * lax.rsqrt(var + EPS)
    return a, be - mean * a


def _acc_stats(i, nt, ne, z2, g_ref, be_ref, a_ref, c_ref, sums_scr):
    upd = jnp.stack([jnp.sum(z2, axis=0), jnp.sum(z2 * z2, axis=0)], axis=0)

    @pl.when(i == 0)
    def _():
        sums_scr[...] = jnp.zeros_like(sums_scr)

    sums_scr[...] += upd

    @pl.when(i == nt - 1)
    def _():
        a, c = _bn_fin(ne, sums_scr[...], g_ref[...], be_ref[...])
        a_ref[...] = a
        c_ref[...] = c


def _s1_body(nt, d, dg, xg_ref, x_ref, w1a_ref, b1_ref, w1b_ref,
             g_ref, be_ref, a_ref, c_ref, sums_scr):
    i = pl.program_id(0)
    h1 = _edge_h1(d, dg, xg_ref, x_ref, w1a_ref, b1_ref, w1b_ref)
    h1f = h1.reshape(KNN * PTILE, h1.shape[-1])
    _acc_stats(i, nt, float(N_EDGE), h1f, g_ref, be_ref, a_ref, c_ref,
               sums_scr)


def _s1(xg, x, d, dg, w1a, b1, w1b, g, be):
    n = x.shape[0]
    h = w1a.shape[1]
    nt = n // PTILE
    return pl.pallas_call(
        functools.partial(_s1_body, nt, d, dg),
        grid=(nt,),
        in_specs=[
            pl.BlockSpec((KNN, PTILE, 128), lambda i: (0, i, 0)),
            pl.BlockSpec((PTILE, 128), lambda i: (i, 0)),
            pl.BlockSpec((d, h), lambda i: (0, 0)),
            pl.BlockSpec((1, h), lambda i: (0, 0)),
            pl.BlockSpec((dg, h), lambda i: (0, 0)),
            pl.BlockSpec((1, h), lambda i: (0, 0)),
            pl.BlockSpec((1, h), lambda i: (0, 0)),
        ],
        out_specs=(pl.BlockSpec((1, h), lambda i: (0, 0)),
                   pl.BlockSpec((1, h), lambda i: (0, 0))),
        out_shape=(jax.ShapeDtypeStruct((1, h), jnp.float32),
                   jax.ShapeDtypeStruct((1, h), jnp.float32)),
        scratch_shapes=[pltpu.VMEM((2, h), jnp.float32)],
    )(xg, x, w1a, b1.reshape(1, h), w1b, g.reshape(1, h), be.reshape(1, h))


def _s2_body(nt, d, dg, xg_ref, x_ref, w1a_ref, b1_ref, w1b_ref,
             a1_ref, c1_ref, w2_ref, b2_ref, g_ref, be_ref,
             a_ref, c_ref, sums_scr):
    i = pl.program_id(0)
    h1 = _edge_h1(d, dg, xg_ref, x_ref, w1a_ref, b1_ref, w1b_ref)
    t = jnp.maximum(h1 * a1_ref[...] + c1_ref[...], 0.0)
    t2 = t.reshape(KNN * PTILE, t.shape[-1])
    z = _mm(t2, w2_ref[...]) + b2_ref[...]
    _acc_stats(i, nt, float(N_EDGE), z, g_ref, be_ref, a_ref, c_ref,
               sums_scr)


def _s2(xg, x, d, dg, w1a, b1, w1b, a1, c1, w2, b2, g2, be2):
    n = x.shape[0]
    h = w1a.shape[1]
    h2 = w2.shape[1]
    nt = n // PTILE
    return pl.pallas_call(
        functools.partial(_s2_body, nt, d, dg),
        grid=(nt,),
        in_specs=[
            pl.BlockSpec((KNN, PTILE, 128), lambda i: (0, i, 0)),
            pl.BlockSpec((PTILE, 128), lambda i: (i, 0)),
            pl.BlockSpec((d, h), lambda i: (0, 0)),
            pl.BlockSpec((1, h), lambda i: (0, 0)),
            pl.BlockSpec((dg, h), lambda i: (0, 0)),
            pl.BlockSpec((1, h), lambda i: (0, 0)),
            pl.BlockSpec((1, h), lambda i: (0, 0)),
            pl.BlockSpec((h, h2), lambda i: (0, 0)),
            pl.BlockSpec((1, h2), lambda i: (0, 0)),
            pl.BlockSpec((1, h2), lambda i: (0, 0)),
            pl.BlockSpec((1, h2), lambda i: (0, 0)),
        ],
        out_specs=(pl.BlockSpec((1, h2), lambda i: (0, 0)),
                   pl.BlockSpec((1, h2), lambda i: (0, 0))),
        out_shape=(jax.ShapeDtypeStruct((1, h2), jnp.float32),
                   jax.ShapeDtypeStruct((1, h2), jnp.float32)),
        scratch_shapes=[pltpu.VMEM((2, h2), jnp.float32)],
    )(xg, x, w1a, b1.reshape(1, h), w1b, a1, c1, w2, b2.reshape(1, h2),
      g2.reshape(1, h2), be2.reshape(1, h2))


def _emax_body(d, dg, xg_ref, x_ref, w1a_ref, b1_ref, w1b_ref,
               a1_ref, c1_ref, w2_ref, b2_ref, a2_ref, c2_ref, out_ref):
    h1 = _edge_h1(d, dg, xg_ref, x_ref, w1a_ref, b1_ref, w1b_ref)
    t = jnp.maximum(h1 * a1_ref[...] + c1_ref[...], 0.0)
    t2 = t.reshape(KNN * PTILE, t.shape[-1])
    z = _mm(t2, w2_ref[...]) + b2_ref[...]
    z = jnp.maximum(z * a2_ref[...] + c2_ref[...], 0.0)
    z3 = z.reshape(KNN, PTILE, z.shape[-1])
    out_ref[...] = jnp.max(z3, axis=0)


def _emax(xg, x, d, dg, w1a, b1, w1b, a1, c1, w2, b2, a2, c2):
    n = x.shape[0]
    h = w1a.shape[1]
    h2 = w2.shape[1]
    nt = n // PTILE
    return pl.pallas_call(
        functools.partial(_emax_body, d, dg),
        grid=(nt,),
        in_specs=[
            pl.BlockSpec((KNN, PTILE, 128), lambda i: (0, i, 0)),
            pl.BlockSpec((PTILE, 128), lambda i: (i, 0)),
            pl.BlockSpec((d, h), lambda i: (0, 0)),
            pl.BlockSpec((1, h), lambda i: (0, 0)),
            pl.BlockSpec((dg, h), lambda i: (0, 0)),
            pl.BlockSpec((1, h), lambda i: (0, 0)),
            pl.BlockSpec((1, h), lambda i: (0, 0)),
            pl.BlockSpec((h, h2), lambda i: (0, 0)),
            pl.BlockSpec((1, h2), lambda i: (0, 0)),
            pl.BlockSpec((1, h2), lambda i: (0, 0)),
            pl.BlockSpec((1, h2), lambda i: (0, 0)),
        ],
        out_specs=pl.BlockSpec((PTILE, h2), lambda i: (i, 0)),
        out_shape=jax.ShapeDtypeStruct((n, h2), jnp.float32),
    )(xg, x, w1a, b1.reshape(1, h), w1b, a1, c1, w2, b2.reshape(1, h2),
      a2, c2)


def _c1_body(nt, x1_ref, x2_ref, w_ref, b_ref, g_ref, be_ref,
             hc_ref, a_ref, c_ref, sums_scr):
    i = pl.program_id(0)
    xc = jnp.concatenate([x1_ref[...], x2_ref[...]], axis=1)
    hc = _mm(xc, w_ref[...]) + b_ref[...]
    hc_ref[...] = hc
    _acc_stats(i, nt, float(N_PTS), hc, g_ref, be_ref, a_ref, c_ref,
               sums_scr)


def _c1(x1, x2, w, b, g, be):
    n = x1.shape[0]
    h1 = x1.shape[1]
    h2 = x2.shape[1]
    ho = w.shape[1]
    nt = n // PTILE
    return pl.pallas_call(
        functools.partial(_c1_body, nt),
        grid=(nt,),
        in_specs=[
            pl.BlockSpec((PTILE, h1), lambda i: (i, 0)),
            pl.BlockSpec((PTILE, h2), lambda i: (i, 0)),
            pl.BlockSpec((h1 + h2, ho), lambda i: (0, 0)),
            pl.BlockSpec((1, ho), lambda i: (0, 0)),
            pl.BlockSpec((1, ho), lambda i: (0, 0)),
            pl.BlockSpec((1, ho), lambda i: (0, 0)),
        ],
        out_specs=(pl.BlockSpec((PTILE, ho), lambda i: (i, 0)),
                   pl.BlockSpec((1, ho), lambda i: (0, 0)),
                   pl.BlockSpec((1, ho), lambda i: (0, 0))),
        out_shape=(jax.ShapeDtypeStruct((n, ho), jnp.float32),
                   jax.ShapeDtypeStruct((1, ho), jnp.float32),
                   jax.ShapeDtypeStruct((1, ho), jnp.float32)),
        scratch_shapes=[pltpu.VMEM((2, ho), jnp.float32)],
    )(x1, x2, w, b.reshape(1, ho), g.reshape(1, ho), be.reshape(1, ho))


def _c2_body(hc_ref, a_ref, c_ref, w2_ref, b2_ref, out_ref):
    t = jnp.maximum(hc_ref[...] * a_ref[...] + c_ref[...], 0.0)
    out_ref[...] = _mm(t, w2_ref[...]) + b2_ref[...]


def _c2(hc, a, c, w2, b2):
    n, h = hc.shape
    nt = n // PTILE
    return pl.pallas_call(
        _c2_body,
        grid=(nt,),
        in_specs=[
            pl.BlockSpec((PTILE, h), lambda i: (i, 0)),
            pl.BlockSpec((1, h), lambda i: (0, 0)),
            pl.BlockSpec((1, h), lambda i: (0, 0)),
            pl.BlockSpec((h, 1), lambda i: (0, 0)),
            pl.BlockSpec((1, 1), lambda i: (0, 0)),
        ],
        out_specs=pl.BlockSpec((PTILE, 1), lambda i: (i, 0)),
        out_shape=jax.ShapeDtypeStruct((n, 1), jnp.float32),
    )(hc, a, c, w2, b2.reshape(1, 1))


def _edgeconv(x, xg_src, dg, bcol, brow, c0s, nchs,
              w1, b1, g1, be1, w2, b2, g2, be2):
    n, d = x.shape
    h = w1.shape[1]
    w1a = w1[:d, :]
    w1b_p = jnp.zeros((dg, h), jnp.float32).at[:d, :].set(w1[d:, :])
    idx = _dist_topk(x, bcol, brow, c0s, nchs)
    idxf = jnp.transpose(idx).reshape(-1)
    xg = _gather_rows(xg_src, idxf).reshape(KNN, n, 128)
    a1, c1 = _s1(xg, xg_src, d, dg, w1a, b1, w1b_p, g1, be1)
    a2, c2 = _s2(xg, xg_src, d, dg, w1a, b1, w1b_p, a1, c1, w2, b2, g2, be2)
    return _emax(xg, xg_src, d, dg, w1a, b1, w1b_p, a1, c1, w2, b2, a2, c2)


def kernel(x, batch,
           c1_w1, c1_b1, c1_g1, c1_be1, c1_w2, c1_b2, c1_g2, c1_be2,
           c2_w1, c2_b1, c2_g1, c2_be1, c2_w2, c2_b2, c2_g2, c2_be2,
           cl_w1, cl_b1, cl_g1, cl_be1, cl_w2, cl_b2):
    n = x.shape[0]
    bi = batch.astype(jnp.int32)
    bcol = bi.reshape(n, 1)
    brow = bi.reshape(1, n)
    seg = jnp.searchsorted(bi, jnp.arange(9, dtype=jnp.int32), side='left')
    seg = seg.astype(jnp.int32)
    rt = jnp.arange(n // ROWS, dtype=jnp.int32)
    blo = bi[rt * ROWS]
    bhi = bi[rt * ROWS + (ROWS - 1)]
    wlo = seg[blo]
    whi = seg[bhi + 1]
    c0s = (wlo // CHUNK) * CHUNK
    nchs = (whi + CHUNK - 1 - c0s) // CHUNK

    x_pad = jnp.zeros((n, 128), jnp.float32).at[:, :x.shape[1]].set(x)
    x1 = _edgeconv(x, x_pad, 16, bcol, brow, c0s, nchs,
                   c1_w1, c1_b1, c1_g1, c1_be1, c1_w2, c1_b2, c1_g2, c1_be2)
    x1_pad = jnp.zeros((n, 128), jnp.float32).at[:, :x1.shape[1]].set(x1)
    x2 = _edgeconv(x1, x1_pad, 64, bcol, brow, c0s, nchs,
                   c2_w1, c2_b1, c2_g1, c2_be1, c2_w2, c2_b2, c2_g2, c2_be2)
    hc, ac, cc = _c1(x1, x2, cl_w1, cl_b1, cl_g1, cl_be1)
    return _c2(hc, ac, cc, cl_w2, cl_b2)
